# baseline (speedup 1.0000x reference)
import functools

import jax
import jax.numpy as jnp
from jax import lax
from jax.experimental import pallas as pl
from jax.experimental.pallas import tpu as pltpu
from jax.experimental.pallas import tpu_sc as plsc

B = 16384
L = 50
D = 64
VOCAB_N = 1000000
N_ROWS = B * L
NC, NS = 2, 16
NW = NC * NS
LP = L // 2
ROWS_PER_W = N_ROWS // NW
CHUNK = 128
N_CHUNKS = ROWS_PER_W // CHUNK

K = 4
NG = N_CHUNKS // K

_sc_mesh = plsc.VectorSubcoreMesh(
    core_axis_name="c", subcore_axis_name="s", num_cores=NC, num_subcores=NS
)


@functools.partial(
    pl.kernel,
    out_type=jax.ShapeDtypeStruct((N_ROWS, D), jnp.bfloat16),
    mesh=_sc_mesh,
    scratch_types=[
        pltpu.VMEM((N_CHUNKS, CHUNK), jnp.int32),
        pltpu.VMEM((N_CHUNKS, CHUNK), jnp.int32),
        [pltpu.VMEM((CHUNK, D), jnp.bfloat16)] * K,
        [pltpu.VMEM((CHUNK, D), jnp.bfloat16)] * K,
        pltpu.SemaphoreType.DMA,
        pltpu.SemaphoreType.DMA,
        pltpu.SemaphoreType.DMA,
        pltpu.SemaphoreType.DMA,
    ],
    compiler_params=pltpu.CompilerParams(use_tc_tiling_on_sc=False),
)
def _sc_gather(table_hbm, idx_hbm, dsti_hbm, out_hbm, idx_v, dsti_v,
               bank0, bank1, sg0, sg1, sc0, sc1):
    wid = lax.axis_index("s") * NC + lax.axis_index("c")
    banks = (bank0, bank1)
    sg = (sg0, sg1)
    sc = (sc0, sc1)
    pltpu.sync_copy(idx_hbm.at[wid], idx_v)
    pltpu.sync_copy(dsti_hbm.at[wid], dsti_v)

    def fire_gathers(g, bk):
        for i in range(K):
            pltpu.async_copy(
                table_hbm.at[idx_v.at[g * K + i]], banks[bk][i], sg[bk]
            )

    def drain(bk, sem_bank):
        for i in range(K):
            pltpu.make_async_copy(
                out_hbm.at[pl.ds(0, CHUNK)], banks[bk][i], sem_bank[bk]
            ).wait()

    def fire_copyouts(g, bk):
        for i in range(K):
            pltpu.async_copy(
                banks[bk][i],
                out_hbm.at[dsti_v.at[g * K + i]],
                sc[bk],
            )

    fire_gathers(0, 0)

    def body(g, carry):
        drain(0, sg)

        @pl.when(g > 0)
        def _():
            drain(1, sc)

        fire_gathers(g + 1, 1)
        fire_copyouts(g, 0)
        drain(1, sg)
        drain(0, sc)

        @pl.when(g + 2 < NG)
        def _():
            fire_gathers(g + 2, 0)

        fire_copyouts(g + 1, 1)
        return carry

    lax.fori_loop(0, NG // 2, lambda t, c: body(t * 2, c), 0)
    drain(1, sc)


SBLK = 16384
NSB = B // SBLK


def _proj_body(e_ref, bd_ref, b2_ref, out_ref):
    e = e_ref[...].reshape(SBLK, 2 * D)
    pt = lax.dot_general(
        bd_ref[...], e, (((0,), (1,)), ((), ())),
        preferred_element_type=jnp.float32,
    ) + b2_ref[...]
    out_ref[...] = pt.reshape(2, D, SBLK)


def _project(emb3, bd, b2):
    return pl.pallas_call(
        _proj_body,
        grid=(LP, NSB),
        in_specs=[
            pl.BlockSpec((1, SBLK, 2 * D), lambda p, j: (p, j, 0)),
            pl.BlockSpec((2 * D, 2 * D), lambda p, j: (0, 0)),
            pl.BlockSpec((2 * D, 1), lambda p, j: (0, 0)),
        ],
        out_specs=pl.BlockSpec((2, D, SBLK), lambda p, j: (p, 0, j)),
        out_shape=jax.ShapeDtypeStruct((L, D, B), jnp.float32),
    )(emb3, bd, b2)


def kernel(x, table, W, b):
    idx3 = x.reshape(NW, N_CHUNKS, CHUNK)
    pat = (jnp.arange(L, dtype=jnp.int32) // 2) * (2 * B) + (
        jnp.arange(L, dtype=jnp.int32) % 2
    )
    dsti = (2 * jnp.arange(B, dtype=jnp.int32))[:, None] + pat[None, :]
    dsti3 = dsti.reshape(NW, N_CHUNKS, CHUNK)
    emb = _sc_gather(table.astype(jnp.bfloat16), idx3, dsti3)
    emb3 = emb.reshape(-1).reshape(LP, B, 2 * D)
    wt = W.T
    bd = (
        jnp.zeros((2 * D, 2 * D), jnp.float32)
        .at[:D, :D].set(wt)
        .at[D:, D:].set(wt)
    ).astype(jnp.bfloat16)
    b2 = jnp.concatenate([b, b]).reshape(2 * D, 1)
    out3 = _project(emb3, bd, b2)
    return jnp.transpose(out3, (2, 0, 1))


# speedup vs baseline: 1.6124x; 1.6124x over previous
import functools

import jax
import jax.numpy as jnp
from jax import lax
from jax.experimental import pallas as pl
from jax.experimental.pallas import tpu as pltpu
from jax.experimental.pallas import tpu_sc as plsc

B = 16384
L = 50
D = 64
N_ROWS = B * L
NC, NS = 2, 16
NW = NC * NS
LP = L // 2
ROWS_PER_W = N_ROWS // NW
CHUNK = 128
N_CHUNKS = ROWS_PER_W // CHUNK

K = 4
NG = N_CHUNKS // K

_sc_mesh = plsc.VectorSubcoreMesh(
    core_axis_name="c", subcore_axis_name="s", num_cores=NC, num_subcores=NS
)


@functools.partial(
    pl.kernel,
    out_type=jax.ShapeDtypeStruct((N_ROWS, D), jnp.float32),
    mesh=_sc_mesh,
    scratch_types=[
        pltpu.VMEM((N_CHUNKS, CHUNK), jnp.int32),
        pltpu.VMEM((N_CHUNKS, CHUNK), jnp.int32),
        [pltpu.VMEM((CHUNK, D), jnp.float32)] * K,
        [pltpu.VMEM((CHUNK, D), jnp.float32)] * K,
        pltpu.SemaphoreType.DMA,
        pltpu.SemaphoreType.DMA,
        pltpu.SemaphoreType.DMA,
        pltpu.SemaphoreType.DMA,
    ],
    compiler_params=pltpu.CompilerParams(use_tc_tiling_on_sc=False),
)
def _sc_gather(table_hbm, idx_hbm, dsti_hbm, out_hbm, idx_v, dsti_v,
               bank0, bank1, sg0, sg1, sc0, sc1):
    wid = lax.axis_index("s") * NC + lax.axis_index("c")
    banks = (bank0, bank1)
    sg = (sg0, sg1)
    sc = (sc0, sc1)
    pltpu.sync_copy(idx_hbm.at[wid], idx_v)
    pltpu.sync_copy(dsti_hbm.at[wid], dsti_v)

    def fire_gathers(g, bk):
        for i in range(K):
            pltpu.async_copy(
                table_hbm.at[idx_v.at[g * K + i]], banks[bk][i], sg[bk]
            )

    def drain(bk, sem_bank):
        for i in range(K):
            pltpu.make_async_copy(
                out_hbm.at[pl.ds(0, CHUNK)], banks[bk][i], sem_bank[bk]
            ).wait()

    def fire_copyouts(g, bk):
        for i in range(K):
            pltpu.async_copy(
                banks[bk][i],
                out_hbm.at[dsti_v.at[g * K + i]],
                sc[bk],
            )

    fire_gathers(0, 0)

    def body(g, carry):
        drain(0, sg)

        @pl.when(g > 0)
        def _():
            drain(1, sc)

        fire_gathers(g + 1, 1)
        fire_copyouts(g, 0)
        drain(1, sg)
        drain(0, sc)

        @pl.when(g + 2 < NG)
        def _():
            fire_gathers(g + 2, 0)

        fire_copyouts(g + 1, 1)
        return carry

    lax.fori_loop(0, NG // 2, lambda t, c: body(t * 2, c), 0)
    drain(1, sc)


SBLK = 16384
NSB = B // SBLK


def _proj_body(e_ref, bd_ref, b2_ref, out_ref):
    e = e_ref[...].reshape(SBLK, 2 * D)
    pt = lax.dot_general(
        bd_ref[...], e, (((0,), (1,)), ((), ())),
        preferred_element_type=jnp.float32,
    ) + b2_ref[...]
    out_ref[...] = pt.reshape(2, D, SBLK)


def _project(emb3, bd, b2):
    return pl.pallas_call(
        _proj_body,
        grid=(LP, NSB),
        in_specs=[
            pl.BlockSpec((1, SBLK, 2 * D), lambda p, j: (p, j, 0)),
            pl.BlockSpec((2 * D, 2 * D), lambda p, j: (0, 0)),
            pl.BlockSpec((2 * D, 1), lambda p, j: (0, 0)),
        ],
        out_specs=pl.BlockSpec((2, D, SBLK), lambda p, j: (p, 0, j)),
        out_shape=jax.ShapeDtypeStruct((L, D, B), jnp.float32),
    )(emb3, bd, b2)


def kernel(x, table, W, b):
    idx3 = x.reshape(NW, N_CHUNKS, CHUNK)
    pat = (jnp.arange(L, dtype=jnp.int32) // 2) * (2 * B) + (
        jnp.arange(L, dtype=jnp.int32) % 2
    )
    dsti = (2 * jnp.arange(B, dtype=jnp.int32))[:, None] + pat[None, :]
    dsti3 = dsti.reshape(NW, N_CHUNKS, CHUNK)
    emb = _sc_gather(table, idx3, dsti3)
    emb3 = emb.reshape(-1).reshape(LP, B, 2 * D)
    wt = W.T
    bd = (
        jnp.zeros((2 * D, 2 * D), jnp.float32)
        .at[:D, :D].set(wt)
        .at[D:, D:].set(wt)
    )
    b2 = jnp.concatenate([b, b]).reshape(2 * D, 1)
    out3 = _project(emb3, bd, b2)
    return jnp.transpose(out3, (2, 0, 1))
